# baseline (speedup 1.0000x reference)
import functools
import math

import jax
import jax.numpy as jnp
import numpy as np
from jax.experimental import pallas as pl
from jax.experimental.pallas import tpu as pltpu

_B, _S, _D = 2, 4096, 2048
_SCOUT, _TOPK = 128, 2048
_SCALE, _BASE, _MAXPOS = 32.0, 1.0e7, 8192

_RA = 512
_RB = 256


def _tables():
    mscale = 0.1 * math.log(_SCALE) + 1.0
    inv = 1.0 / (_BASE ** (np.arange(0, _SCOUT, 2, dtype=np.float64) / _SCOUT))
    mid = _SCOUT // 2
    inv = np.concatenate([inv[:mid] / _SCALE, inv[mid:]])
    t = np.arange(_MAXPOS, dtype=np.float64) / _SCALE
    freqs = np.outer(t, inv)
    emb = np.concatenate([freqs, freqs], axis=-1)
    m = mscale * mscale
    return (np.cos(emb) * m).astype(np.float32), (np.sin(emb) * m).astype(np.float32)


_COS_T, _SIN_T = _tables()


def _rotate_half(x):
    h = x.shape[-1] // 2
    return jnp.concatenate([-x[..., h:], x[..., :h]], axis=-1)


def _proj_rope_kernel(h_ref, pos_ref, wq_ref, wk_ref, cos_ref, sin_ref,
                      q_ref, k_ref):
    h = h_ref[...]
    pos = pos_ref[...]
    q = jax.lax.dot_general(h, wq_ref[...], (((1,), (1,)), ((), ())),
                            preferred_element_type=jnp.float32)
    k = jax.lax.dot_general(h, wk_ref[...], (((1,), (1,)), ((), ())),
                            preferred_element_type=jnp.float32)
    col = jax.lax.broadcasted_iota(jnp.int32, (_RA, _S), 1)
    oh = (col == pos).astype(jnp.float32)
    cos = jax.lax.dot_general(oh, cos_ref[...], (((1,), (0,)), ((), ())),
                              precision=jax.lax.Precision.HIGHEST,
                              preferred_element_type=jnp.float32)
    sin = jax.lax.dot_general(oh, sin_ref[...], (((1,), (0,)), ((), ())),
                              precision=jax.lax.Precision.HIGHEST,
                              preferred_element_type=jnp.float32)
    q_ref[...] = q * cos + _rotate_half(q) * sin
    k_ref[...] = k * cos + _rotate_half(k) * sin


def _topk_kernel(q_ref, k_ref, out_ref):
    q = q_ref[0]
    k = k_ref[0]
    s = jax.lax.dot_general(q, k, (((1,), (1,)), ((), ())),
                            preferred_element_type=jnp.float32)
    s = s / math.sqrt(_SCOUT)
    r0 = pl.program_id(1) * _RB
    col = jax.lax.broadcasted_iota(jnp.int32, (_RB, _S), 1)
    row = jax.lax.broadcasted_iota(jnp.int32, (_RB, _S), 0) + r0
    colf = col.astype(jnp.float32)
    key = jnp.where(col <= row, s, -(colf + 1.0) * 1e30)
    idx = col
    n = _S
    log_n = n.bit_length() - 1
    for pk in range(1, log_n + 1):
        kk = 1 << pk
        for pj in range(pk - 1, -1, -1):
            jj = 1 << pj
            is_left = (col & jj) == 0
            take_max = jnp.logical_not(
                jnp.logical_xor(is_left, (col & kk) == 0))
            other_key = jnp.where(is_left,
                                  pltpu.roll(key, n - jj, 1),
                                  pltpu.roll(key, jj, 1))
            other_idx = jnp.where(is_left,
                                  pltpu.roll(idx, n - jj, 1),
                                  pltpu.roll(idx, jj, 1))
            swap = ((take_max & (other_key > key))
                    | (jnp.logical_not(take_max) & (other_key < key)))
            key = jnp.where(swap, other_key, key)
            idx = jnp.where(swap, other_idx, idx)
    out_ref[0] = idx[:, :_TOPK]


@jax.jit
def kernel(hidden_states, position_ids, Wq, Wk):
    bsz, seq_len, _ = hidden_states.shape
    h2 = hidden_states.reshape(bsz * seq_len, _D)
    pos2 = position_ids.reshape(bsz * seq_len, 1).astype(jnp.int32)
    cos_t = jnp.asarray(_COS_T[:seq_len])
    sin_t = jnp.asarray(_SIN_T[:seq_len])
    nblk = (bsz * seq_len) // _RA
    q2, k2 = pl.pallas_call(
        _proj_rope_kernel,
        grid=(nblk,),
        in_specs=[
            pl.BlockSpec((_RA, _D), lambda i: (i, 0)),
            pl.BlockSpec((_RA, 1), lambda i: (i, 0)),
            pl.BlockSpec((_SCOUT, _D), lambda i: (0, 0)),
            pl.BlockSpec((_SCOUT, _D), lambda i: (0, 0)),
            pl.BlockSpec((_S, _SCOUT), lambda i: (0, 0)),
            pl.BlockSpec((_S, _SCOUT), lambda i: (0, 0)),
        ],
        out_specs=[
            pl.BlockSpec((_RA, _SCOUT), lambda i: (i, 0)),
            pl.BlockSpec((_RA, _SCOUT), lambda i: (i, 0)),
        ],
        out_shape=[
            jax.ShapeDtypeStruct((bsz * seq_len, _SCOUT), jnp.float32),
            jax.ShapeDtypeStruct((bsz * seq_len, _SCOUT), jnp.float32),
        ],
    )(h2, pos2, Wq, Wk, cos_t, sin_t)

    q3 = q2.reshape(bsz, seq_len, _SCOUT)
    k3 = k2.reshape(bsz, seq_len, _SCOUT)

    topk = pl.pallas_call(
        _topk_kernel,
        grid=(bsz, seq_len // _RB),
        in_specs=[
            pl.BlockSpec((1, _RB, _SCOUT), lambda b, r: (b, r, 0)),
            pl.BlockSpec((1, _S, _SCOUT), lambda b, r: (b, 0, 0)),
        ],
        out_specs=pl.BlockSpec((1, _RB, _TOPK), lambda b, r: (b, r, 0)),
        out_shape=jax.ShapeDtypeStruct((bsz, seq_len, _TOPK), jnp.int32),
    )(q3, k3)
    return topk, k3


# speedup vs baseline: 1.6600x; 1.6600x over previous
import functools
import math

import jax
import jax.numpy as jnp
import numpy as np
from jax.experimental import pallas as pl
from jax.experimental.pallas import tpu as pltpu

_B, _S, _D = 2, 4096, 2048
_SCOUT, _TOPK = 128, 2048
_SCALE, _BASE, _MAXPOS = 32.0, 1.0e7, 8192
_HALF = _S // 2

_RA = 512
_RB = 256


def _tables():
    mscale = 0.1 * math.log(_SCALE) + 1.0
    inv = 1.0 / (_BASE ** (np.arange(0, _SCOUT, 2, dtype=np.float64) / _SCOUT))
    mid = _SCOUT // 2
    inv = np.concatenate([inv[:mid] / _SCALE, inv[mid:]])
    t = np.arange(_MAXPOS, dtype=np.float64) / _SCALE
    freqs = np.outer(t, inv)
    emb = np.concatenate([freqs, freqs], axis=-1)
    m = mscale * mscale
    return (np.cos(emb) * m).astype(np.float32), (np.sin(emb) * m).astype(np.float32)


_COS_T, _SIN_T = _tables()


def _rotate_half(x):
    h = x.shape[-1] // 2
    return jnp.concatenate([-x[..., h:], x[..., :h]], axis=-1)


def _proj_rope_kernel(h_ref, pos_ref, wq_ref, wk_ref, cos_ref, sin_ref,
                      q_ref, k_ref):
    h = h_ref[...]
    pos = pos_ref[...]
    q = jax.lax.dot_general(h, wq_ref[...], (((1,), (1,)), ((), ())),
                            preferred_element_type=jnp.float32)
    k = jax.lax.dot_general(h, wk_ref[...], (((1,), (1,)), ((), ())),
                            preferred_element_type=jnp.float32)
    col = jax.lax.broadcasted_iota(jnp.int32, (_RA, _S), 1)
    oh = (col == pos).astype(jnp.float32)
    cos = jax.lax.dot_general(oh, cos_ref[...], (((1,), (0,)), ((), ())),
                              precision=jax.lax.Precision.HIGHEST,
                              preferred_element_type=jnp.float32)
    sin = jax.lax.dot_general(oh, sin_ref[...], (((1,), (0,)), ((), ())),
                              precision=jax.lax.Precision.HIGHEST,
                              preferred_element_type=jnp.float32)
    q_ref[...] = q * cos + _rotate_half(q) * sin
    k_ref[...] = k * cos + _rotate_half(k) * sin


def _stage(key, idx, col1, take_max1, jj, n):
    is_left1 = (col1 & jj) == 0
    okey = jnp.where(is_left1, pltpu.roll(key, n - jj, 1),
                     pltpu.roll(key, jj, 1))
    oidx = jnp.where(is_left1, pltpu.roll(idx, n - jj, 1),
                     pltpu.roll(idx, jj, 1))
    nkey = jnp.where(take_max1, jnp.maximum(key, okey),
                     jnp.minimum(key, okey))
    moved = nkey != key
    nidx = jnp.where(moved, oidx, idx)
    return nkey, nidx


def _sort_halves(key, idx, col1, n, flip_high):
    half_mask1 = col1 >= (n // 2)
    coll1 = col1 & (n // 2 - 1)
    log_h = (n // 2).bit_length() - 1
    for pk in range(1, log_h + 1):
        kk = 1 << pk
        for pj in range(pk - 1, -1, -1):
            jj = 1 << pj
            is_left1 = (col1 & jj) == 0
            desc1 = (coll1 & kk) == 0
            take_max1 = is_left1 == desc1
            if flip_high:
                take_max1 = take_max1 != half_mask1
            key, idx = _stage(key, idx, col1, take_max1, jj, n)
    return key, idx


def _merge_desc(key, idx, col1, n):
    jj = n // 2
    while jj >= 1:
        is_left1 = (col1 & jj) == 0
        key, idx = _stage(key, idx, col1, is_left1, jj, n)
        jj //= 2
    return key, idx


def _topk_low_kernel(q_ref, k_ref, out_ref):
    q = q_ref[0]
    k = k_ref[0]
    s = jax.lax.dot_general(q, k, (((1,), (1,)), ((), ())),
                            preferred_element_type=jnp.float32)
    s = s / math.sqrt(_SCOUT)
    r0 = pl.program_id(1) * _RB
    col1 = jax.lax.broadcasted_iota(jnp.int32, (1, _HALF), 1)
    row = jax.lax.broadcasted_iota(jnp.int32, (_RB, 1), 0) + r0
    sent1 = -(col1.astype(jnp.float32) + 1.0) * 1e30
    key = jnp.where(col1 <= row, s, sent1)
    idx = jnp.broadcast_to(col1, (_RB, _HALF))
    key, idx = _sort_halves(key, idx, col1, _HALF, flip_high=True)
    key, idx = _merge_desc(key, idx, col1, _HALF)
    out_ref[0] = idx


def _topk_high_kernel(q_ref, k_ref, out_ref):
    q = q_ref[0]
    k = k_ref[0]
    s = jax.lax.dot_general(q, k, (((1,), (1,)), ((), ())),
                            preferred_element_type=jnp.float32)
    s = s / math.sqrt(_SCOUT)
    r0 = _HALF + pl.program_id(1) * _RB
    col1 = jax.lax.broadcasted_iota(jnp.int32, (1, _S), 1)
    row = jax.lax.broadcasted_iota(jnp.int32, (_RB, 1), 0) + r0
    sent1 = -(col1.astype(jnp.float32) + 1.0) * 1e30
    key = jnp.where(col1 <= row, s, sent1)
    idx = jnp.broadcast_to(col1, (_RB, _S))
    key, idx = _sort_halves(key, idx, col1, _S, flip_high=True)
    a_key, b_key = key[:, :_HALF], key[:, _HALF:]
    a_idx, b_idx = idx[:, :_HALF], idx[:, _HALF:]
    ckey = jnp.maximum(a_key, b_key)
    moved = ckey != a_key
    cidx = jnp.where(moved, b_idx, a_idx)
    colh1 = jax.lax.broadcasted_iota(jnp.int32, (1, _HALF), 1)
    ckey, cidx = _merge_desc(ckey, cidx, colh1, _HALF)
    out_ref[0] = cidx


@jax.jit
def kernel(hidden_states, position_ids, Wq, Wk):
    bsz, seq_len, _ = hidden_states.shape
    h2 = hidden_states.reshape(bsz * seq_len, _D)
    pos2 = position_ids.reshape(bsz * seq_len, 1).astype(jnp.int32)
    cos_t = jnp.asarray(_COS_T[:seq_len])
    sin_t = jnp.asarray(_SIN_T[:seq_len])
    nblk = (bsz * seq_len) // _RA
    q2, k2 = pl.pallas_call(
        _proj_rope_kernel,
        grid=(nblk,),
        in_specs=[
            pl.BlockSpec((_RA, _D), lambda i: (i, 0)),
            pl.BlockSpec((_RA, 1), lambda i: (i, 0)),
            pl.BlockSpec((_SCOUT, _D), lambda i: (0, 0)),
            pl.BlockSpec((_SCOUT, _D), lambda i: (0, 0)),
            pl.BlockSpec((_S, _SCOUT), lambda i: (0, 0)),
            pl.BlockSpec((_S, _SCOUT), lambda i: (0, 0)),
        ],
        out_specs=[
            pl.BlockSpec((_RA, _SCOUT), lambda i: (i, 0)),
            pl.BlockSpec((_RA, _SCOUT), lambda i: (i, 0)),
        ],
        out_shape=[
            jax.ShapeDtypeStruct((bsz * seq_len, _SCOUT), jnp.float32),
            jax.ShapeDtypeStruct((bsz * seq_len, _SCOUT), jnp.float32),
        ],
    )(h2, pos2, Wq, Wk, cos_t, sin_t)

    q3 = q2.reshape(bsz, seq_len, _SCOUT)
    k3 = k2.reshape(bsz, seq_len, _SCOUT)

    topk_low = pl.pallas_call(
        _topk_low_kernel,
        grid=(bsz, _HALF // _RB),
        in_specs=[
            pl.BlockSpec((1, _RB, _SCOUT), lambda b, r: (b, r, 0)),
            pl.BlockSpec((1, _HALF, _SCOUT), lambda b, r: (b, 0, 0)),
        ],
        out_specs=pl.BlockSpec((1, _RB, _TOPK), lambda b, r: (b, r, 0)),
        out_shape=jax.ShapeDtypeStruct((bsz, _HALF, _TOPK), jnp.int32),
    )(q3, k3)

    topk_high = pl.pallas_call(
        _topk_high_kernel,
        grid=(bsz, _HALF // _RB),
        in_specs=[
            pl.BlockSpec((1, _RB, _SCOUT), lambda b, r: (b, r + _HALF // _RB, 0)),
            pl.BlockSpec((1, _S, _SCOUT), lambda b, r: (b, 0, 0)),
        ],
        out_specs=pl.BlockSpec((1, _RB, _TOPK), lambda b, r: (b, r, 0)),
        out_shape=jax.ShapeDtypeStruct((bsz, _HALF, _TOPK), jnp.int32),
    )(q3, k3)

    topk = jnp.concatenate([topk_low, topk_high], axis=1)
    return topk, k3


# speedup vs baseline: 1.9631x; 1.1826x over previous
import functools
import math

import jax
import jax.numpy as jnp
import numpy as np
from jax.experimental import pallas as pl
from jax.experimental.pallas import tpu as pltpu

_B, _S, _D = 2, 4096, 2048
_SCOUT, _TOPK = 128, 2048
_SCALE, _BASE, _MAXPOS = 32.0, 1.0e7, 8192
_HALF = _S // 2

_RA = 512
_RB = 256


def _tables():
    mscale = 0.1 * math.log(_SCALE) + 1.0
    inv = 1.0 / (_BASE ** (np.arange(0, _SCOUT, 2, dtype=np.float64) / _SCOUT))
    mid = _SCOUT // 2
    inv = np.concatenate([inv[:mid] / _SCALE, inv[mid:]])
    t = np.arange(_MAXPOS, dtype=np.float64) / _SCALE
    freqs = np.outer(t, inv)
    emb = np.concatenate([freqs, freqs], axis=-1)
    m = mscale * mscale
    return (np.cos(emb) * m).astype(np.float32), (np.sin(emb) * m).astype(np.float32)


_COS_T, _SIN_T = _tables()


def _rotate_half(x):
    h = x.shape[-1] // 2
    return jnp.concatenate([-x[..., h:], x[..., :h]], axis=-1)


def _proj_rope_kernel(h_ref, pos_ref, wq_ref, wk_ref, cos_ref, sin_ref,
                      q_ref, k_ref):
    h = h_ref[...]
    pos = pos_ref[...]
    q = jax.lax.dot_general(h, wq_ref[...], (((1,), (1,)), ((), ())),
                            preferred_element_type=jnp.float32)
    k = jax.lax.dot_general(h, wk_ref[...], (((1,), (1,)), ((), ())),
                            preferred_element_type=jnp.float32)
    col = jax.lax.broadcasted_iota(jnp.int32, (_RA, _S), 1)
    oh = (col == pos).astype(jnp.float32)
    cos = jax.lax.dot_general(oh, cos_ref[...], (((1,), (0,)), ((), ())),
                              precision=jax.lax.Precision.HIGHEST,
                              preferred_element_type=jnp.float32)
    sin = jax.lax.dot_general(oh, sin_ref[...], (((1,), (0,)), ((), ())),
                              precision=jax.lax.Precision.HIGHEST,
                              preferred_element_type=jnp.float32)
    q_ref[...] = q * cos + _rotate_half(q) * sin
    k_ref[...] = k * cos + _rotate_half(k) * sin


def _stage(key, idx, col1, take_max1, jj, n):
    is_left1 = (col1 & jj) == 0
    okey = jnp.where(is_left1, pltpu.roll(key, n - jj, 1),
                     pltpu.roll(key, jj, 1))
    oidx = jnp.where(is_left1, pltpu.roll(idx, n - jj, 1),
                     pltpu.roll(idx, jj, 1))
    nkey = jnp.where(take_max1, jnp.maximum(key, okey),
                     jnp.minimum(key, okey))
    moved = nkey != key
    nidx = jnp.where(moved, oidx, idx)
    return nkey, nidx


def _sort_halves(key, idx, col1, n, flip_high):
    half_mask1 = col1 >= (n // 2)
    coll1 = col1 & (n // 2 - 1)
    log_h = (n // 2).bit_length() - 1
    for pk in range(1, log_h + 1):
        kk = 1 << pk
        for pj in range(pk - 1, -1, -1):
            jj = 1 << pj
            is_left1 = (col1 & jj) == 0
            desc1 = (coll1 & kk) == 0
            take_max1 = is_left1 == desc1
            if flip_high:
                take_max1 = take_max1 != half_mask1
            key, idx = _stage(key, idx, col1, take_max1, jj, n)
    return key, idx


def _merge_desc(key, idx, col1, n):
    jj = n // 2
    while jj >= 1:
        is_left1 = (col1 & jj) == 0
        key, idx = _stage(key, idx, col1, is_left1, jj, n)
        jj //= 2
    return key, idx


def _sort_desc_full(key, idx, col1, w, ascending=False):
    log_w = w.bit_length() - 1
    for pk in range(1, log_w + 1):
        kk = 1 << pk
        for pj in range(pk - 1, -1, -1):
            jj = 1 << pj
            is_left1 = (col1 & jj) == 0
            desc1 = (col1 & kk) == 0 if kk < w else (col1 >= 0)
            take_max1 = is_left1 == desc1
            if ascending:
                take_max1 = jnp.logical_not(take_max1)
            key, idx = _stage(key, idx, col1, take_max1, jj, w)
    return key, idx


def _topk_low_kernel(w, r_base, q_ref, k_ref, out_ref):
    q = q_ref[0]
    k = k_ref[0]
    s = jax.lax.dot_general(q, k, (((1,), (1,)), ((), ())),
                            preferred_element_type=jnp.float32)
    s = s / math.sqrt(_SCOUT)
    r0 = r_base + pl.program_id(1) * _RB
    col1 = jax.lax.broadcasted_iota(jnp.int32, (1, w), 1)
    row = jax.lax.broadcasted_iota(jnp.int32, (_RB, 1), 0) + r0
    sent1 = -(col1.astype(jnp.float32) + 1.0) * 1e30
    key = jnp.where(col1 <= row, s, sent1)
    idx = jnp.broadcast_to(col1, (_RB, w))
    key, idx = _sort_halves(key, idx, col1, w, flip_high=True)
    key, idx = _merge_desc(key, idx, col1, w)
    if w < _TOPK:
        tail1 = jax.lax.broadcasted_iota(jnp.int32, (1, _TOPK - w), 1) + w
        idx = jnp.concatenate(
            [idx, jnp.broadcast_to(tail1, (_RB, _TOPK - w))], axis=1)
    out_ref[0] = idx


def _topk_high_kernel(wb, r_base, q_ref, ka_ref, kb_ref, out_ref):
    q = q_ref[0]
    ka = ka_ref[0]
    kb = kb_ref[0]
    sa = jax.lax.dot_general(q, ka, (((1,), (1,)), ((), ())),
                             preferred_element_type=jnp.float32)
    sa = sa / math.sqrt(_SCOUT)
    sb = jax.lax.dot_general(q, kb, (((1,), (1,)), ((), ())),
                             preferred_element_type=jnp.float32)
    sb = sb / math.sqrt(_SCOUT)
    r0 = r_base + pl.program_id(1) * _RB
    row = jax.lax.broadcasted_iota(jnp.int32, (_RB, 1), 0) + r0
    cola1 = jax.lax.broadcasted_iota(jnp.int32, (1, _HALF), 1)
    colb1 = jax.lax.broadcasted_iota(jnp.int32, (1, wb), 1)
    colbg1 = colb1 + _HALF
    a_key = sa
    a_idx = jnp.broadcast_to(cola1, (_RB, _HALF))
    sentb1 = -(colbg1.astype(jnp.float32) + 1.0) * 1e30
    b_key = jnp.where(colbg1 <= row, sb, sentb1)
    b_idx = jnp.broadcast_to(colbg1, (_RB, wb))
    a_key, a_idx = _sort_halves(a_key, a_idx, cola1, _HALF, flip_high=True)
    a_key, a_idx = _merge_desc(a_key, a_idx, cola1, _HALF)
    b_key, b_idx = _sort_desc_full(b_key, b_idx, colb1, wb, ascending=True)
    if wb < _HALF:
        pw = _HALF - wb
        pre1 = jax.lax.broadcasted_iota(jnp.int32, (1, pw), 1)
        prek1 = (pre1.astype(jnp.float32) - 8192.0) * 1e30
        b_key = jnp.concatenate(
            [jnp.broadcast_to(prek1, (_RB, pw)), b_key], axis=1)
        b_idx = jnp.concatenate(
            [jnp.broadcast_to(pre1, (_RB, pw)), b_idx], axis=1)
    ckey = jnp.maximum(a_key, b_key)
    moved = ckey != a_key
    cidx = jnp.where(moved, b_idx, a_idx)
    ckey, cidx = _merge_desc(ckey, cidx, cola1, _HALF)
    out_ref[0] = cidx


@jax.jit
def kernel(hidden_states, position_ids, Wq, Wk):
    bsz, seq_len, _ = hidden_states.shape
    h2 = hidden_states.reshape(bsz * seq_len, _D)
    pos2 = position_ids.reshape(bsz * seq_len, 1).astype(jnp.int32)
    cos_t = jnp.asarray(_COS_T[:seq_len])
    sin_t = jnp.asarray(_SIN_T[:seq_len])
    nblk = (bsz * seq_len) // _RA
    q2, k2 = pl.pallas_call(
        _proj_rope_kernel,
        grid=(nblk,),
        in_specs=[
            pl.BlockSpec((_RA, _D), lambda i: (i, 0)),
            pl.BlockSpec((_RA, 1), lambda i: (i, 0)),
            pl.BlockSpec((_SCOUT, _D), lambda i: (0, 0)),
            pl.BlockSpec((_SCOUT, _D), lambda i: (0, 0)),
            pl.BlockSpec((_S, _SCOUT), lambda i: (0, 0)),
            pl.BlockSpec((_S, _SCOUT), lambda i: (0, 0)),
        ],
        out_specs=[
            pl.BlockSpec((_RA, _SCOUT), lambda i: (i, 0)),
            pl.BlockSpec((_RA, _SCOUT), lambda i: (i, 0)),
        ],
        out_shape=[
            jax.ShapeDtypeStruct((bsz * seq_len, _SCOUT), jnp.float32),
            jax.ShapeDtypeStruct((bsz * seq_len, _SCOUT), jnp.float32),
        ],
    )(h2, pos2, Wq, Wk, cos_t, sin_t)

    q3 = q2.reshape(bsz, seq_len, _SCOUT)
    k3 = k2.reshape(bsz, seq_len, _SCOUT)

    quarter = _S // 4
    nq = quarter // _RB

    def low_call(w, r_base):
        qoff = r_base // _RB
        return pl.pallas_call(
            functools.partial(_topk_low_kernel, w, r_base),
            grid=(bsz, nq),
            in_specs=[
                pl.BlockSpec((1, _RB, _SCOUT), lambda b, r: (b, r + qoff, 0)),
                pl.BlockSpec((1, w, _SCOUT), lambda b, r: (b, 0, 0)),
            ],
            out_specs=pl.BlockSpec((1, _RB, _TOPK), lambda b, r: (b, r, 0)),
            out_shape=jax.ShapeDtypeStruct((bsz, quarter, _TOPK), jnp.int32),
        )(q3, k3)

    def high_call(wb, r_base):
        qoff = r_base // _RB
        return pl.pallas_call(
            functools.partial(_topk_high_kernel, wb, r_base),
            grid=(bsz, nq),
            in_specs=[
                pl.BlockSpec((1, _RB, _SCOUT), lambda b, r: (b, r + qoff, 0)),
                pl.BlockSpec((1, _HALF, _SCOUT), lambda b, r: (b, 0, 0)),
                pl.BlockSpec((1, wb, _SCOUT),
                             lambda b, r: (b, _HALF // wb, 0)),
            ],
            out_specs=pl.BlockSpec((1, _RB, _TOPK), lambda b, r: (b, r, 0)),
            out_shape=jax.ShapeDtypeStruct((bsz, quarter, _TOPK), jnp.int32),
        )(q3, k3, k3)

    topk = jnp.concatenate([
        low_call(1024, 0),
        low_call(2048, quarter),
        high_call(1024, 2 * quarter),
        high_call(2048, 3 * quarter),
    ], axis=1)
    return topk, k3


# speedup vs baseline: 2.0233x; 1.0307x over previous
import functools
import math

import jax
import jax.numpy as jnp
import numpy as np
from jax.experimental import pallas as pl
from jax.experimental.pallas import tpu as pltpu

_B, _S, _D = 2, 4096, 2048
_SCOUT, _TOPK = 128, 2048
_SCALE, _BASE, _MAXPOS = 32.0, 1.0e7, 8192
_HALF = _S // 2

_RA = 512
_RB = 256


def _tables():
    mscale = 0.1 * math.log(_SCALE) + 1.0
    inv = 1.0 / (_BASE ** (np.arange(0, _SCOUT, 2, dtype=np.float64) / _SCOUT))
    mid = _SCOUT // 2
    inv = np.concatenate([inv[:mid] / _SCALE, inv[mid:]])
    t = np.arange(_MAXPOS, dtype=np.float64) / _SCALE
    freqs = np.outer(t, inv)
    emb = np.concatenate([freqs, freqs], axis=-1)
    m = mscale * mscale
    return (np.cos(emb) * m).astype(np.float32), (np.sin(emb) * m).astype(np.float32)


_COS_T, _SIN_T = _tables()


def _rotate_half(x):
    h = x.shape[-1] // 2
    return jnp.concatenate([-x[..., h:], x[..., :h]], axis=-1)


def _proj_rope_kernel(h_ref, pos_ref, wq_ref, wk_ref, cos_ref, sin_ref,
                      q_ref, k_ref):
    h = h_ref[...]
    pos = pos_ref[...]
    q = jax.lax.dot_general(h, wq_ref[...], (((1,), (1,)), ((), ())),
                            preferred_element_type=jnp.float32)
    k = jax.lax.dot_general(h, wk_ref[...], (((1,), (1,)), ((), ())),
                            preferred_element_type=jnp.float32)
    col = jax.lax.broadcasted_iota(jnp.int32, (_RA, _S), 1)
    oh = (col == pos).astype(jnp.float32)
    cos = jax.lax.dot_general(oh, cos_ref[...], (((1,), (0,)), ((), ())),
                              precision=jax.lax.Precision.HIGHEST,
                              preferred_element_type=jnp.float32)
    sin = jax.lax.dot_general(oh, sin_ref[...], (((1,), (0,)), ((), ())),
                              precision=jax.lax.Precision.HIGHEST,
                              preferred_element_type=jnp.float32)
    q_ref[...] = q * cos + _rotate_half(q) * sin
    k_ref[...] = k * cos + _rotate_half(k) * sin


def _stage(key, idx, col1, take_max1, jj, n):
    is_left1 = (col1 & jj) == 0
    okey = jnp.where(is_left1, pltpu.roll(key, n - jj, 1),
                     pltpu.roll(key, jj, 1))
    oidx = jnp.where(is_left1, pltpu.roll(idx, n - jj, 1),
                     pltpu.roll(idx, jj, 1))
    nkey = jnp.where(take_max1, jnp.maximum(key, okey),
                     jnp.minimum(key, okey))
    moved = nkey != key
    nidx = jnp.where(moved, oidx, idx)
    return nkey, nidx


def _sort_halves(key, idx, col1, n, flip_high):
    half_mask1 = col1 >= (n // 2)
    coll1 = col1 & (n // 2 - 1)
    log_h = (n // 2).bit_length() - 1
    for pk in range(1, log_h + 1):
        kk = 1 << pk
        for pj in range(pk - 1, -1, -1):
            jj = 1 << pj
            is_left1 = (col1 & jj) == 0
            desc1 = (coll1 & kk) == 0
            take_max1 = is_left1 == desc1
            if flip_high:
                take_max1 = take_max1 != half_mask1
            key, idx = _stage(key, idx, col1, take_max1, jj, n)
    return key, idx


def _merge_desc(key, idx, col1, n):
    jj = n // 2
    while jj >= 1:
        is_left1 = (col1 & jj) == 0
        key, idx = _stage(key, idx, col1, is_left1, jj, n)
        jj //= 2
    return key, idx


def _sort_desc_full(key, idx, col1, w, ascending=False):
    log_w = w.bit_length() - 1
    for pk in range(1, log_w + 1):
        kk = 1 << pk
        for pj in range(pk - 1, -1, -1):
            jj = 1 << pj
            is_left1 = (col1 & jj) == 0
            desc1 = (col1 & kk) == 0 if kk < w else (col1 >= 0)
            take_max1 = is_left1 == desc1
            if ascending:
                take_max1 = jnp.logical_not(take_max1)
            key, idx = _stage(key, idx, col1, take_max1, jj, w)
    return key, idx


def _topk_low_kernel(w, r_base, q_ref, k_ref, out_ref):
    q = q_ref[0]
    k = k_ref[0]
    s = jax.lax.dot_general(q, k, (((1,), (1,)), ((), ())),
                            preferred_element_type=jnp.float32)
    s = s / math.sqrt(_SCOUT)
    r0 = r_base + pl.program_id(1) * _RB
    col1 = jax.lax.broadcasted_iota(jnp.int32, (1, w), 1)
    row = jax.lax.broadcasted_iota(jnp.int32, (_RB, 1), 0) + r0
    sent1 = -(col1.astype(jnp.float32) + 1.0) * 1e30
    key = jnp.where(col1 <= row, s, sent1)
    idx = jnp.broadcast_to(col1, (_RB, w))
    key, idx = _sort_halves(key, idx, col1, w, flip_high=True)
    key, idx = _merge_desc(key, idx, col1, w)
    if w < _TOPK:
        tail1 = jax.lax.broadcasted_iota(jnp.int32, (1, _TOPK - w), 1) + w
        idx = jnp.concatenate(
            [idx, jnp.broadcast_to(tail1, (_RB, _TOPK - w))], axis=1)
    out_ref[0] = idx


def _topk_high_kernel(wb, r_base, q_ref, ka_ref, kb_ref, out_ref):
    q = q_ref[0]
    ka = ka_ref[0]
    kb = kb_ref[0]
    sa = jax.lax.dot_general(q, ka, (((1,), (1,)), ((), ())),
                             preferred_element_type=jnp.float32)
    sa = sa / math.sqrt(_SCOUT)
    sb = jax.lax.dot_general(q, kb, (((1,), (1,)), ((), ())),
                             preferred_element_type=jnp.float32)
    sb = sb / math.sqrt(_SCOUT)
    r0 = r_base + pl.program_id(1) * _RB
    row = jax.lax.broadcasted_iota(jnp.int32, (_RB, 1), 0) + r0
    cola1 = jax.lax.broadcasted_iota(jnp.int32, (1, _HALF), 1)
    colb1 = jax.lax.broadcasted_iota(jnp.int32, (1, wb), 1)
    colbg1 = colb1 + _HALF
    a_key = sa
    a_idx = jnp.broadcast_to(cola1, (_RB, _HALF))
    sentb1 = -(colbg1.astype(jnp.float32) + 1.0) * 1e30
    b_key = jnp.where(colbg1 <= row, sb, sentb1)
    b_idx = jnp.broadcast_to(colbg1, (_RB, wb))
    a_key, a_idx = _sort_halves(a_key, a_idx, cola1, _HALF, flip_high=True)
    a_key, a_idx = _merge_desc(a_key, a_idx, cola1, _HALF)
    b_key, b_idx = _sort_desc_full(b_key, b_idx, colb1, wb, ascending=True)
    if wb < _HALF:
        pw = _HALF - wb
        pre1 = jax.lax.broadcasted_iota(jnp.int32, (1, pw), 1)
        prek1 = (pre1.astype(jnp.float32) - 8192.0) * 1e30
        b_key = jnp.concatenate(
            [jnp.broadcast_to(prek1, (_RB, pw)), b_key], axis=1)
        b_idx = jnp.concatenate(
            [jnp.broadcast_to(pre1, (_RB, pw)), b_idx], axis=1)
    ckey = jnp.maximum(a_key, b_key)
    moved = ckey != a_key
    cidx = jnp.where(moved, b_idx, a_idx)
    ckey, cidx = _merge_desc(ckey, cidx, cola1, _HALF)
    out_ref[0] = cidx


@jax.jit
def kernel(hidden_states, position_ids, Wq, Wk):
    bsz, seq_len, _ = hidden_states.shape
    h2 = hidden_states.reshape(bsz * seq_len, _D)
    pos2 = position_ids.reshape(bsz * seq_len, 1).astype(jnp.int32)
    cos_t = jnp.asarray(_COS_T[:seq_len])
    sin_t = jnp.asarray(_SIN_T[:seq_len])
    nblk = (bsz * seq_len) // _RA
    q2, k2 = pl.pallas_call(
        _proj_rope_kernel,
        grid=(nblk,),
        in_specs=[
            pl.BlockSpec((_RA, _D), lambda i: (i, 0)),
            pl.BlockSpec((_RA, 1), lambda i: (i, 0)),
            pl.BlockSpec((_SCOUT, _D), lambda i: (0, 0)),
            pl.BlockSpec((_SCOUT, _D), lambda i: (0, 0)),
            pl.BlockSpec((_S, _SCOUT), lambda i: (0, 0)),
            pl.BlockSpec((_S, _SCOUT), lambda i: (0, 0)),
        ],
        out_specs=[
            pl.BlockSpec((_RA, _SCOUT), lambda i: (i, 0)),
            pl.BlockSpec((_RA, _SCOUT), lambda i: (i, 0)),
        ],
        out_shape=[
            jax.ShapeDtypeStruct((bsz * seq_len, _SCOUT), jnp.float32),
            jax.ShapeDtypeStruct((bsz * seq_len, _SCOUT), jnp.float32),
        ],
    )(h2, pos2, Wq, Wk, cos_t, sin_t)

    q3 = q2.reshape(bsz, seq_len, _SCOUT)
    k3 = k2.reshape(bsz, seq_len, _SCOUT)

    quarter = _S // 8
    nq = quarter // _RB

    def low_call(w, r_base):
        qoff = r_base // _RB
        return pl.pallas_call(
            functools.partial(_topk_low_kernel, w, r_base),
            grid=(bsz, nq),
            in_specs=[
                pl.BlockSpec((1, _RB, _SCOUT), lambda b, r: (b, r + qoff, 0)),
                pl.BlockSpec((1, w, _SCOUT), lambda b, r: (b, 0, 0)),
            ],
            out_specs=pl.BlockSpec((1, _RB, _TOPK), lambda b, r: (b, r, 0)),
            out_shape=jax.ShapeDtypeStruct((bsz, quarter, _TOPK), jnp.int32),
        )(q3, k3)

    def high_call(wb, r_base):
        qoff = r_base // _RB
        return pl.pallas_call(
            functools.partial(_topk_high_kernel, wb, r_base),
            grid=(bsz, nq),
            in_specs=[
                pl.BlockSpec((1, _RB, _SCOUT), lambda b, r: (b, r + qoff, 0)),
                pl.BlockSpec((1, _HALF, _SCOUT), lambda b, r: (b, 0, 0)),
                pl.BlockSpec((1, wb, _SCOUT),
                             lambda b, r: (b, _HALF // wb, 0)),
            ],
            out_specs=pl.BlockSpec((1, _RB, _TOPK), lambda b, r: (b, r, 0)),
            out_shape=jax.ShapeDtypeStruct((bsz, quarter, _TOPK), jnp.int32),
        )(q3, k3, k3)

    topk = jnp.concatenate([
        low_call(512, 0),
        low_call(1024, quarter),
        low_call(2048, 2 * quarter),
        low_call(2048, 3 * quarter),
        high_call(512, 4 * quarter),
        high_call(1024, 5 * quarter),
        high_call(2048, 6 * quarter),
        high_call(2048, 7 * quarter),
    ], axis=1)
    return topk, k3
